# Initial kernel scaffold; baseline (speedup 1.0000x reference)
import jax
import jax.numpy as jnp
from jax.experimental import pallas as pl


def kernel(x, edge_index, batch, W1_0, b1_0, W2_0, b2_0, W1_1, b1_1, W2_1, b2_1, W1_2, b1_2, W2_2, b2_2):
    raise NotImplementedError("write your pallas kernel here")


# speedup vs baseline: 1.0381x; 1.0381x over previous
import functools

import jax
import jax.numpy as jnp
from jax.experimental import pallas as pl

N = 10000
E = 160000
F_IN = 256
DIM = 512
L = 3
G = 64

BM = 2000


def _mlp_body(z_ref, w1_ref, b1_ref, w2_ref, b2_ref, o_ref):
    t = jnp.dot(z_ref[...], w1_ref[...], preferred_element_type=jnp.float32)
    t = jnp.maximum(t + b1_ref[...], 0.0)
    h = jnp.dot(t, w2_ref[...], preferred_element_type=jnp.float32)
    o_ref[...] = jnp.maximum(h + b2_ref[...], 0.0)


def _mlp(z, w1, b1, w2, b2):
    din = z.shape[1]
    grid = N // BM
    return pl.pallas_call(
        _mlp_body,
        grid=(grid,),
        in_specs=[
            pl.BlockSpec((BM, din), lambda i: (i, 0)),
            pl.BlockSpec((din, DIM), lambda i: (0, 0)),
            pl.BlockSpec((1, DIM), lambda i: (0, 0)),
            pl.BlockSpec((DIM, DIM), lambda i: (0, 0)),
            pl.BlockSpec((1, DIM), lambda i: (0, 0)),
        ],
        out_specs=pl.BlockSpec((BM, DIM), lambda i: (i, 0)),
        out_shape=jax.ShapeDtypeStruct((N, DIM), jnp.float32),
    )(z, w1, b1.reshape(1, DIM), w2, b2.reshape(1, DIM))


def _mlp_pool_body(z_ref, w1_ref, b1_ref, w2_ref, b2_ref, batch_ref, o_ref):
    t = jnp.dot(z_ref[...], w1_ref[...], preferred_element_type=jnp.float32)
    t = jnp.maximum(t + b1_ref[...], 0.0)
    h = jnp.dot(t, w2_ref[...], preferred_element_type=jnp.float32)
    h = jnp.maximum(h + b2_ref[...], 0.0)
    bids = batch_ref[0]
    onehot = (jax.lax.broadcasted_iota(jnp.int32, (G, BM), 0) == bids).astype(
        jnp.float32)
    contrib = jnp.dot(onehot, h, preferred_element_type=jnp.float32)

    @pl.when(pl.program_id(0) == 0)
    def _():
        o_ref[...] = contrib

    @pl.when(pl.program_id(0) != 0)
    def _():
        o_ref[...] += contrib


def _mlp_pool(z, w1, b1, w2, b2, batch3):
    din = z.shape[1]
    grid = N // BM
    return pl.pallas_call(
        _mlp_pool_body,
        grid=(grid,),
        in_specs=[
            pl.BlockSpec((BM, din), lambda i: (i, 0)),
            pl.BlockSpec((din, DIM), lambda i: (0, 0)),
            pl.BlockSpec((1, DIM), lambda i: (0, 0)),
            pl.BlockSpec((DIM, DIM), lambda i: (0, 0)),
            pl.BlockSpec((1, DIM), lambda i: (0, 0)),
            pl.BlockSpec((1, 1, BM), lambda i: (i, 0, 0)),
        ],
        out_specs=pl.BlockSpec((G, DIM), lambda i: (0, 0)),
        out_shape=jax.ShapeDtypeStruct((G, DIM), jnp.float32),
    )(z, w1, b1.reshape(1, DIM), w2, b2.reshape(1, DIM), batch3)


def _aggregate(h, src, dst):
    agg = jnp.zeros((N, h.shape[1]), jnp.float32).at[dst].add(h[src])
    return h + agg


def kernel(x, edge_index, batch,
           W1_0, b1_0, W2_0, b2_0,
           W1_1, b1_1, W2_1, b2_1,
           W1_2, b1_2, W2_2, b2_2):
    src = edge_index[0].astype(jnp.int32)
    dst = edge_index[1].astype(jnp.int32)
    batch3 = batch.astype(jnp.int32).reshape(N // BM, 1, BM)

    h = x
    z = _aggregate(h, src, dst)
    h = _mlp(z, W1_0, b1_0, W2_0, b2_0)
    z = _aggregate(h, src, dst)
    h = _mlp(z, W1_1, b1_1, W2_1, b2_1)
    z = _aggregate(h, src, dst)
    return _mlp_pool(z, W1_2, b1_2, W2_2, b2_2, batch3)


# speedup vs baseline: 1.1152x; 1.0742x over previous
import functools

import jax
import jax.numpy as jnp
from jax import lax
from jax.experimental import pallas as pl
from jax.experimental.pallas import tpu as pltpu
from jax.experimental.pallas import tpu_sc as plsc

N = 10000
E = 160000
F_IN = 256
DIM = 512
G = 64

BM = 2000

NC = 2
NS = 16
LANES = 16

WROWS = 160
NWIN = 63
NROUND = 2
ACC_ROWS = 168
NOUT = NWIN * WROWS
CHUNK = 4000
NCHK = E // CHUNK
GRP = CHUNK // LANES
GB = 16
CCAP = CHUNK + LANES


def _sc_agg_body(h_hbm, src_hbm, dst_hbm, out_hbm,
                 srcb, dstb, csrc, cldst, rows, acc, semg):
    c = lax.axis_index("c")
    s = lax.axis_index("s")
    wid = c * NS + s
    d = rows.shape[1]
    zero = jnp.zeros((LANES,), jnp.float32)
    lanesv = lax.iota(jnp.int32, LANES)

    for r in range(NROUND):
        w = r * (NC * NS) + wid

        @pl.when(w < NWIN)
        def _():
            base = w * WROWS

            def zbody(i, carry):
                for j in range(d // LANES):
                    acc[i, pl.ds(j * LANES, LANES)] = zero
                return carry

            lax.fori_loop(0, ACC_ROWS, zbody, 0)

            def chunk_body(k, carry):
                pltpu.sync_copy(dst_hbm.at[pl.ds(k * CHUNK, CHUNK)], dstb)
                pltpu.sync_copy(src_hbm.at[pl.ds(k * CHUNK, CHUNK)], srcb)

                def fbody(i, nv):
                    dv = dstb[pl.ds(i * LANES, LANES)]
                    sv = srcb[pl.ds(i * LANES, LANES)]
                    m = (dv >= base) & (dv < base + WROWS)
                    mi = jnp.where(m, 1, 0)
                    pos = nv + plsc.cumsum(mi) - 1
                    plsc.store_scatter(csrc, [pos], sv, mask=m)
                    plsc.store_scatter(cldst, [pos], dv - base, mask=m)
                    return nv + plsc.all_reduce_population_count(m)

                nv = lax.fori_loop(0, GRP, fbody,
                                   jnp.zeros((LANES,), jnp.int32))
                n = nv[0]

                g0 = n // LANES
                keep = lanesv < (n - g0 * LANES)
                vs = csrc[pl.ds(g0 * LANES, LANES)]
                csrc[pl.ds(g0 * LANES, LANES)] = jnp.where(keep, vs, 0)
                vd = cldst[pl.ds(g0 * LANES, LANES)]
                cldst[pl.ds(g0 * LANES, LANES)] = jnp.where(keep, vd, WROWS)
                nb = (n + GB - 1) // GB

                def gbody(b, carry):
                    pltpu.async_copy(h_hbm.at[csrc.at[pl.ds(b * GB, GB)]],
                                     rows, semg).wait()
                    ldstv = cldst[pl.ds(b * GB, GB)]
                    for kk in range(GB):
                        row = ldstv[kk]
                        for j in range(d // LANES):
                            plsc.addupdate(
                                acc.at[row, pl.ds(j * LANES, LANES)],
                                rows[kk, pl.ds(j * LANES, LANES)])
                    return carry

                lax.fori_loop(0, nb, gbody, 0)
                return carry

            lax.fori_loop(0, NCHK, chunk_body, 0)

            pltpu.sync_copy(acc.at[pl.ds(0, WROWS)],
                            out_hbm.at[pl.ds(base, WROWS)])


def _aggregate(h, src, dst):
    d = h.shape[1]
    mesh = plsc.VectorSubcoreMesh(core_axis_name="c", subcore_axis_name="s",
                                  num_cores=NC, num_subcores=NS)
    k = pl.kernel(
        _sc_agg_body,
        out_type=jax.ShapeDtypeStruct((NOUT, d), jnp.float32),
        mesh=mesh,
        compiler_params=pltpu.CompilerParams(needs_layout_passes=False),
        scratch_types=[
            pltpu.VMEM((CHUNK,), jnp.int32),
            pltpu.VMEM((CHUNK,), jnp.int32),
            pltpu.VMEM((CCAP,), jnp.int32),
            pltpu.VMEM((CCAP,), jnp.int32),
            pltpu.VMEM((GB, d), jnp.float32),
            pltpu.VMEM((ACC_ROWS, d), jnp.float32),
            pltpu.SemaphoreType.DMA,
        ],
    )
    return k(h, src, dst)[:N]


def _mlp_body(h_ref, agg_ref, w1_ref, b1_ref, w2_ref, b2_ref, o_ref):
    z = h_ref[...] + agg_ref[...]
    t = jnp.dot(z, w1_ref[...], preferred_element_type=jnp.float32)
    t = jnp.maximum(t + b1_ref[...], 0.0)
    o = jnp.dot(t, w2_ref[...], preferred_element_type=jnp.float32)
    o_ref[...] = jnp.maximum(o + b2_ref[...], 0.0)


def _mlp(h, agg, w1, b1, w2, b2):
    din = h.shape[1]
    grid = N // BM
    return pl.pallas_call(
        _mlp_body,
        grid=(grid,),
        in_specs=[
            pl.BlockSpec((BM, din), lambda i: (i, 0)),
            pl.BlockSpec((BM, din), lambda i: (i, 0)),
            pl.BlockSpec((din, DIM), lambda i: (0, 0)),
            pl.BlockSpec((1, DIM), lambda i: (0, 0)),
            pl.BlockSpec((DIM, DIM), lambda i: (0, 0)),
            pl.BlockSpec((1, DIM), lambda i: (0, 0)),
        ],
        out_specs=pl.BlockSpec((BM, DIM), lambda i: (i, 0)),
        out_shape=jax.ShapeDtypeStruct((N, DIM), jnp.float32),
    )(h, agg, w1, b1.reshape(1, DIM), w2, b2.reshape(1, DIM))


def _mlp_pool_body(h_ref, agg_ref, w1_ref, b1_ref, w2_ref, b2_ref,
                   batch_ref, o_ref):
    z = h_ref[...] + agg_ref[...]
    t = jnp.dot(z, w1_ref[...], preferred_element_type=jnp.float32)
    t = jnp.maximum(t + b1_ref[...], 0.0)
    o = jnp.dot(t, w2_ref[...], preferred_element_type=jnp.float32)
    o = jnp.maximum(o + b2_ref[...], 0.0)
    bids = batch_ref[0]
    onehot = (jax.lax.broadcasted_iota(jnp.int32, (G, BM), 0) == bids).astype(
        jnp.float32)
    contrib = jnp.dot(onehot, o, preferred_element_type=jnp.float32)

    @pl.when(pl.program_id(0) == 0)
    def _():
        o_ref[...] = contrib

    @pl.when(pl.program_id(0) != 0)
    def _():
        o_ref[...] += contrib


def _mlp_pool(h, agg, w1, b1, w2, b2, batch3):
    din = h.shape[1]
    grid = N // BM
    return pl.pallas_call(
        _mlp_pool_body,
        grid=(grid,),
        in_specs=[
            pl.BlockSpec((BM, din), lambda i: (i, 0)),
            pl.BlockSpec((BM, din), lambda i: (i, 0)),
            pl.BlockSpec((din, DIM), lambda i: (0, 0)),
            pl.BlockSpec((1, DIM), lambda i: (0, 0)),
            pl.BlockSpec((DIM, DIM), lambda i: (0, 0)),
            pl.BlockSpec((1, DIM), lambda i: (0, 0)),
            pl.BlockSpec((1, 1, BM), lambda i: (i, 0, 0)),
        ],
        out_specs=pl.BlockSpec((G, DIM), lambda i: (0, 0)),
        out_shape=jax.ShapeDtypeStruct((G, DIM), jnp.float32),
    )(h, agg, w1, b1.reshape(1, DIM), w2, b2.reshape(1, DIM), batch3)


def kernel(x, edge_index, batch,
           W1_0, b1_0, W2_0, b2_0,
           W1_1, b1_1, W2_1, b2_1,
           W1_2, b1_2, W2_2, b2_2):
    src = edge_index[0].astype(jnp.int32)
    dst = edge_index[1].astype(jnp.int32)
    batch3 = batch.astype(jnp.int32).reshape(N // BM, 1, BM)

    h = x
    agg = _aggregate(h, src, dst)
    h = _mlp(h, agg, W1_0, b1_0, W2_0, b2_0)
    agg = _aggregate(h, src, dst)
    h = _mlp(h, agg, W1_1, b1_1, W2_1, b2_1)
    agg = _aggregate(h, src, dst)
    return _mlp_pool(h, agg, W1_2, b1_2, W2_2, b2_2, batch3)


# speedup vs baseline: 1.2922x; 1.1587x over previous
import functools

import jax
import jax.numpy as jnp
from jax import lax
from jax.experimental import pallas as pl
from jax.experimental.pallas import tpu as pltpu
from jax.experimental.pallas import tpu_sc as plsc

N = 10000
E = 160000
F_IN = 256
DIM = 512
G = 64

BM = 2000

NC = 2
NS = 16
LANES = 16

WROWS = 160
NWIN = 63
NROUND = 2
ACC_ROWS = 161
NOUT = NWIN * WROWS
CHUNK = 3200
NCHK = E // CHUNK
GRP = CHUNK // LANES
GB = 16
PIPE = 3
CCAP = CHUNK + LANES


def _sc_agg_body(h_hbm, src_hbm, dst_hbm, out_hbm,
                 srcb, dstb, csrc, cldst, rows, acc, semg, seme):
    c = lax.axis_index("c")
    s = lax.axis_index("s")
    wid = c * NS + s
    d = rows.shape[1]
    zero = jnp.zeros((LANES,), jnp.float32)
    lanesv = lax.iota(jnp.int32, LANES)

    for r in range(NROUND):
        w = r * (NC * NS) + wid

        @pl.when(w < NWIN)
        def _():
            base = w * WROWS

            def zbody(i, carry):
                for j in range(d // LANES):
                    acc[i, pl.ds(j * LANES, LANES)] = zero
                return carry

            lax.fori_loop(0, ACC_ROWS, zbody, 0)

            def chunk_body(k, carry):
                cpd = pltpu.async_copy(dst_hbm.at[pl.ds(k * CHUNK, CHUNK)],
                                       dstb, seme)
                cps = pltpu.async_copy(src_hbm.at[pl.ds(k * CHUNK, CHUNK)],
                                       srcb, seme)
                cpd.wait()
                cps.wait()

                def fbody(i, nv):
                    dv = dstb[pl.ds(i * LANES, LANES)]
                    sv = srcb[pl.ds(i * LANES, LANES)]
                    m = (dv >= base) & (dv < base + WROWS)
                    mi = jnp.where(m, 1, 0)
                    pos = nv + plsc.cumsum(mi) - 1
                    plsc.store_scatter(csrc, [pos], sv, mask=m)
                    plsc.store_scatter(cldst, [pos], dv - base, mask=m)
                    return nv + plsc.all_reduce_population_count(m)

                nv = lax.fori_loop(0, GRP, fbody,
                                   jnp.zeros((LANES,), jnp.int32))
                n = nv[0]

                g0 = n // LANES
                keep = lanesv < (n - g0 * LANES)
                vs = csrc[pl.ds(g0 * LANES, LANES)]
                csrc[pl.ds(g0 * LANES, LANES)] = jnp.where(keep, vs, 0)
                vd = cldst[pl.ds(g0 * LANES, LANES)]
                cldst[pl.ds(g0 * LANES, LANES)] = jnp.where(keep, vd, WROWS)
                nb = (n + GB - 1) // GB

                def gcopy(b):
                    return pltpu.make_async_copy(
                        h_hbm.at[csrc.at[pl.ds(b * GB, GB)]],
                        rows.at[pl.ds((b % PIPE) * GB, GB)], semg)

                def prebody(b, carry):
                    gcopy(b).start()
                    return carry

                lax.fori_loop(0, jnp.minimum(nb, PIPE), prebody, 0)

                def gbody(b, carry):
                    gcopy(b).wait()

                    @pl.when(b + PIPE < nb)
                    def _():
                        gcopy(b + PIPE).start()

                    ldstv = cldst[pl.ds(b * GB, GB)]
                    off = (b % PIPE) * GB
                    for kk in range(GB):
                        row = ldstv[kk]
                        for j in range(d // LANES):
                            plsc.addupdate(
                                acc.at[row, pl.ds(j * LANES, LANES)],
                                rows[off + kk, pl.ds(j * LANES, LANES)])
                    return carry

                lax.fori_loop(0, nb, gbody, 0)
                return carry

            lax.fori_loop(0, NCHK, chunk_body, 0)

            pltpu.sync_copy(acc.at[pl.ds(0, WROWS)],
                            out_hbm.at[pl.ds(base, WROWS)])


def _aggregate(h, src, dst):
    d = h.shape[1]
    mesh = plsc.VectorSubcoreMesh(core_axis_name="c", subcore_axis_name="s",
                                  num_cores=NC, num_subcores=NS)
    k = pl.kernel(
        _sc_agg_body,
        out_type=jax.ShapeDtypeStruct((NOUT, d), jnp.float32),
        mesh=mesh,
        compiler_params=pltpu.CompilerParams(needs_layout_passes=False),
        scratch_types=[
            pltpu.VMEM((CHUNK,), jnp.int32),
            pltpu.VMEM((CHUNK,), jnp.int32),
            pltpu.VMEM((CCAP,), jnp.int32),
            pltpu.VMEM((CCAP,), jnp.int32),
            pltpu.VMEM((PIPE * GB, d), jnp.float32),
            pltpu.VMEM((ACC_ROWS, d), jnp.float32),
            pltpu.SemaphoreType.DMA,
            pltpu.SemaphoreType.DMA,
        ],
    )
    return k(h, src, dst)[:N]


def _mlp_body(h_ref, agg_ref, w1_ref, b1_ref, w2_ref, b2_ref, o_ref):
    z = h_ref[...] + agg_ref[...]
    t = jnp.dot(z, w1_ref[...], preferred_element_type=jnp.float32)
    t = jnp.maximum(t + b1_ref[...], 0.0)
    o = jnp.dot(t, w2_ref[...], preferred_element_type=jnp.float32)
    o_ref[...] = jnp.maximum(o + b2_ref[...], 0.0)


def _mlp(h, agg, w1, b1, w2, b2):
    din = h.shape[1]
    grid = N // BM
    return pl.pallas_call(
        _mlp_body,
        grid=(grid,),
        in_specs=[
            pl.BlockSpec((BM, din), lambda i: (i, 0)),
            pl.BlockSpec((BM, din), lambda i: (i, 0)),
            pl.BlockSpec((din, DIM), lambda i: (0, 0)),
            pl.BlockSpec((1, DIM), lambda i: (0, 0)),
            pl.BlockSpec((DIM, DIM), lambda i: (0, 0)),
            pl.BlockSpec((1, DIM), lambda i: (0, 0)),
        ],
        out_specs=pl.BlockSpec((BM, DIM), lambda i: (i, 0)),
        out_shape=jax.ShapeDtypeStruct((N, DIM), jnp.float32),
    )(h, agg, w1, b1.reshape(1, DIM), w2, b2.reshape(1, DIM))


def _mlp_pool_body(h_ref, agg_ref, w1_ref, b1_ref, w2_ref, b2_ref,
                   batch_ref, o_ref):
    z = h_ref[...] + agg_ref[...]
    t = jnp.dot(z, w1_ref[...], preferred_element_type=jnp.float32)
    t = jnp.maximum(t + b1_ref[...], 0.0)
    o = jnp.dot(t, w2_ref[...], preferred_element_type=jnp.float32)
    o = jnp.maximum(o + b2_ref[...], 0.0)
    bids = batch_ref[0]
    onehot = (jax.lax.broadcasted_iota(jnp.int32, (G, BM), 0) == bids).astype(
        jnp.float32)
    contrib = jnp.dot(onehot, o, preferred_element_type=jnp.float32)

    @pl.when(pl.program_id(0) == 0)
    def _():
        o_ref[...] = contrib

    @pl.when(pl.program_id(0) != 0)
    def _():
        o_ref[...] += contrib


def _mlp_pool(h, agg, w1, b1, w2, b2, batch3):
    din = h.shape[1]
    grid = N // BM
    return pl.pallas_call(
        _mlp_pool_body,
        grid=(grid,),
        in_specs=[
            pl.BlockSpec((BM, din), lambda i: (i, 0)),
            pl.BlockSpec((BM, din), lambda i: (i, 0)),
            pl.BlockSpec((din, DIM), lambda i: (0, 0)),
            pl.BlockSpec((1, DIM), lambda i: (0, 0)),
            pl.BlockSpec((DIM, DIM), lambda i: (0, 0)),
            pl.BlockSpec((1, DIM), lambda i: (0, 0)),
            pl.BlockSpec((1, 1, BM), lambda i: (i, 0, 0)),
        ],
        out_specs=pl.BlockSpec((G, DIM), lambda i: (0, 0)),
        out_shape=jax.ShapeDtypeStruct((G, DIM), jnp.float32),
    )(h, agg, w1, b1.reshape(1, DIM), w2, b2.reshape(1, DIM), batch3)


def kernel(x, edge_index, batch,
           W1_0, b1_0, W2_0, b2_0,
           W1_1, b1_1, W2_1, b2_1,
           W1_2, b1_2, W2_2, b2_2):
    src = edge_index[0].astype(jnp.int32)
    dst = edge_index[1].astype(jnp.int32)
    batch3 = batch.astype(jnp.int32).reshape(N // BM, 1, BM)

    h = x
    agg = _aggregate(h, src, dst)
    h = _mlp(h, agg, W1_0, b1_0, W2_0, b2_0)
    agg = _aggregate(h, src, dst)
    h = _mlp(h, agg, W1_1, b1_1, W2_1, b2_1)
    agg = _aggregate(h, src, dst)
    return _mlp_pool(h, agg, W1_2, b1_2, W2_2, b2_2, batch3)
